# Initial kernel scaffold; baseline (speedup 1.0000x reference)
import jax
import jax.numpy as jnp
from jax.experimental import pallas as pl


def kernel(obs, enc_W1, enc_b1, enc_W2, enc_b2, c1_Wq, c1_bq, c1_Wk, c1_bk, c1_Wv, c1_bv, c2_Wq, c2_bq, c2_Wk, c2_bk, c2_Wv, c2_bv, out_W, out_b):
    raise NotImplementedError("write your pallas kernel here")


# speedup vs baseline: 17.7671x; 17.7671x over previous
import functools
import math

import jax
import jax.numpy as jnp
from jax.experimental import pallas as pl

BS = 64
N = 500
NP = 512
INPUT_DIM = 6
HIDDEN = 32
HEADS = 2
OUT_DIM = 10
RADIUS = 0.1
D_MODEL = HIDDEN * HEADS
NEG = -jnp.inf


def _attn(x, mask_add, Wq, bq, Wk, bk, Wv, bv):
    scale = 1.0 / math.sqrt(HIDDEN)
    q = (x @ Wq + bq) * scale
    k = x @ Wk + bk
    v = x @ Wv + bv
    outs = []
    for h in range(HEADS):
        sl = slice(h * HIDDEN, (h + 1) * HIDDEN)
        qh = q[:, sl]
        kh = k[:, sl]
        vh = v[:, sl]
        al = jax.lax.dot_general(
            qh, kh, (((1,), (1,)), ((), ())),
            preferred_element_type=jnp.float32,
        )
        al = al + mask_add
        amax = jnp.max(al, axis=1, keepdims=True)
        amax = jnp.maximum(amax, -1e30)
        ex = jnp.exp(al - amax)
        den = jnp.sum(ex, axis=1, keepdims=True) + 1e-16
        oh = jax.lax.dot_general(
            ex, vh, (((1,), (0,)), ((), ())),
            preferred_element_type=jnp.float32,
        )
        outs.append(oh / den)
    return jnp.concatenate(outs, axis=1)


def _body(feats_ref, pos_ref, posT_ref, onehot_ref,
          W1_ref, b1_ref, W2_ref, b2_ref,
          q1_ref, bq1_ref, k1_ref, bk1_ref, v1_ref, bv1_ref,
          q2_ref, bq2_ref, k2_ref, bk2_ref, v2_ref, bv2_ref,
          oW_ref, ob_ref, out_ref):
    feats = feats_ref[0]
    h = jnp.maximum(feats @ W1_ref[:] + b1_ref[:], 0.0)
    h = jnp.maximum(h @ W2_ref[:] + b2_ref[:], 0.0)

    pos = pos_ref[0]
    posT = posT_ref[0]
    dx = pos[:, 0:1] - posT[0:1, :]
    dy = pos[:, 1:2] - posT[1:2, :]
    d2 = dx * dx + dy * dy
    ii = jax.lax.broadcasted_iota(jnp.int32, (NP, NP), 0)
    jj = jax.lax.broadcasted_iota(jnp.int32, (NP, NP), 1)
    allowed = (d2 < RADIUS * RADIUS) & (ii != jj)
    mask_add = jnp.where(allowed, 0.0, NEG)

    h = jnp.maximum(
        _attn(h, mask_add, q1_ref[:], bq1_ref[:], k1_ref[:], bk1_ref[:],
              v1_ref[:], bv1_ref[:]), 0.0)
    h = jnp.maximum(
        _attn(h, mask_add, q2_ref[:], bq2_ref[:], k2_ref[:], bk2_ref[:],
              v2_ref[:], bv2_ref[:]), 0.0)

    emb = jax.lax.dot_general(
        onehot_ref[0], h, (((1,), (0,)), ((), ())),
        preferred_element_type=jnp.float32,
    )
    out_ref[0] = emb @ oW_ref[:] + ob_ref[:]


@jax.jit
def kernel(obs, enc_W1, enc_b1, enc_W2, enc_b2,
           c1_Wq, c1_bq, c1_Wk, c1_bk, c1_Wv, c1_bv,
           c2_Wq, c2_bq, c2_Wk, c2_bk, c2_Wv, c2_bv,
           out_W, out_b):
    node = obs[:, :N * (2 + INPUT_DIM)].reshape(BS, N, 2 + INPUT_DIM)
    pos = node[:, :, :2]
    feats = node[:, :, 2:]
    pos_p = jnp.pad(pos, ((0, 0), (0, NP - N), (0, 0)), constant_values=1e6)
    feats_p = jnp.pad(feats, ((0, 0), (0, NP - N), (0, 2)))
    posT_p = jnp.swapaxes(pos_p, 1, 2)

    agent = jnp.clip(obs[:, -1], 0, N - 1).astype(jnp.int32)
    onehot = jax.nn.one_hot(agent, NP, dtype=jnp.float32)[:, None, :]

    W1 = jnp.pad(enc_W1, ((0, 2), (0, 0)))
    b1 = enc_b1[None, :]
    b2 = enc_b2[None, :]
    bq1 = c1_bq[None, :]; bk1 = c1_bk[None, :]; bv1 = c1_bv[None, :]
    bq2 = c2_bq[None, :]; bk2 = c2_bk[None, :]; bv2 = c2_bv[None, :]
    ob = out_b[None, :]

    def fixed(shape):
        nd = len(shape)
        return pl.BlockSpec(shape, lambda b: (0,) * nd)

    in_specs = [
        pl.BlockSpec((1, NP, 8), lambda b: (b, 0, 0)),
        pl.BlockSpec((1, NP, 2), lambda b: (b, 0, 0)),
        pl.BlockSpec((1, 2, NP), lambda b: (b, 0, 0)),
        pl.BlockSpec((1, 1, NP), lambda b: (b, 0, 0)),
        fixed((8, HIDDEN)), fixed((1, HIDDEN)),
        fixed((HIDDEN, HIDDEN)), fixed((1, HIDDEN)),
        fixed((HIDDEN, D_MODEL)), fixed((1, D_MODEL)),
        fixed((HIDDEN, D_MODEL)), fixed((1, D_MODEL)),
        fixed((HIDDEN, D_MODEL)), fixed((1, D_MODEL)),
        fixed((D_MODEL, D_MODEL)), fixed((1, D_MODEL)),
        fixed((D_MODEL, D_MODEL)), fixed((1, D_MODEL)),
        fixed((D_MODEL, D_MODEL)), fixed((1, D_MODEL)),
        fixed((D_MODEL, OUT_DIM)), fixed((1, OUT_DIM)),
    ]

    out = pl.pallas_call(
        _body,
        grid=(BS,),
        in_specs=in_specs,
        out_specs=pl.BlockSpec((1, 1, OUT_DIM), lambda b: (b, 0, 0)),
        out_shape=jax.ShapeDtypeStruct((BS, 1, OUT_DIM), jnp.float32),
    )(feats_p, pos_p, posT_p, onehot,
      W1, b1, enc_W2, b2,
      c1_Wq, bq1, c1_Wk, bk1, c1_Wv, bv1,
      c2_Wq, bq2, c2_Wk, bk2, c2_Wv, bv2,
      out_W, ob)
    return out[:, 0, :]


# speedup vs baseline: 22.5440x; 1.2689x over previous
import functools
import math

import jax
import jax.numpy as jnp
from jax.experimental import pallas as pl

BS = 64
N = 500
NP = 512
INPUT_DIM = 6
HIDDEN = 32
HEADS = 2
OUT_DIM = 10
RADIUS = 0.1
D_MODEL = HIDDEN * HEADS
NEG = -jnp.inf


def _attn(x, mask_add, Wq, bq, Wk, bk, Wv, bv):
    scale = 1.0 / math.sqrt(HIDDEN)
    q = (x @ Wq + bq) * scale
    k = x @ Wk + bk
    v = x @ Wv + bv
    ones = jnp.ones((NP, 1), jnp.bfloat16)
    outs = []
    for h in range(HEADS):
        sl = slice(h * HIDDEN, (h + 1) * HIDDEN)
        qh = q[:, sl].astype(jnp.bfloat16)
        kh = k[:, sl].astype(jnp.bfloat16)
        vh = v[:, sl].astype(jnp.bfloat16)
        al = jax.lax.dot_general(
            qh, kh, (((1,), (1,)), ((), ())),
            preferred_element_type=jnp.float32,
        )
        al = al + mask_add
        amax = jnp.max(al, axis=1, keepdims=True)
        amax = jnp.maximum(amax, -1e30)
        ex = jnp.exp(al - amax).astype(jnp.bfloat16)
        vext = jnp.concatenate([vh, ones], axis=1)
        oh = jax.lax.dot_general(
            ex, vext, (((1,), (0,)), ((), ())),
            preferred_element_type=jnp.float32,
        )
        den = oh[:, HIDDEN:HIDDEN + 1] + 1e-16
        outs.append(oh[:, :HIDDEN] / den)
    return jnp.concatenate(outs, axis=1)


def _body(feats_ref, pos_ref, posT_ref, onehot_ref,
          W1_ref, b1_ref, W2_ref, b2_ref,
          q1_ref, bq1_ref, k1_ref, bk1_ref, v1_ref, bv1_ref,
          q2_ref, bq2_ref, k2_ref, bk2_ref, v2_ref, bv2_ref,
          oW_ref, ob_ref, out_ref):
    feats = feats_ref[0]
    h = jnp.maximum(feats @ W1_ref[:] + b1_ref[:], 0.0)
    h = jnp.maximum(h @ W2_ref[:] + b2_ref[:], 0.0)

    pos = pos_ref[0]
    posT = posT_ref[0]
    dx = pos[:, 0:1] - posT[0:1, :]
    dy = pos[:, 1:2] - posT[1:2, :]
    d2 = dx * dx + dy * dy
    ii = jax.lax.broadcasted_iota(jnp.int32, (NP, NP), 0)
    jj = jax.lax.broadcasted_iota(jnp.int32, (NP, NP), 1)
    allowed = (d2 < RADIUS * RADIUS) & (ii != jj)
    mask_add = jnp.where(allowed, 0.0, NEG)

    h = jnp.maximum(
        _attn(h, mask_add, q1_ref[:], bq1_ref[:], k1_ref[:], bk1_ref[:],
              v1_ref[:], bv1_ref[:]), 0.0)
    h = jnp.maximum(
        _attn(h, mask_add, q2_ref[:], bq2_ref[:], k2_ref[:], bk2_ref[:],
              v2_ref[:], bv2_ref[:]), 0.0)

    emb = jax.lax.dot_general(
        onehot_ref[0], h, (((1,), (0,)), ((), ())),
        preferred_element_type=jnp.float32,
    )
    out_ref[0] = emb @ oW_ref[:] + ob_ref[:]


@jax.jit
def kernel(obs, enc_W1, enc_b1, enc_W2, enc_b2,
           c1_Wq, c1_bq, c1_Wk, c1_bk, c1_Wv, c1_bv,
           c2_Wq, c2_bq, c2_Wk, c2_bk, c2_Wv, c2_bv,
           out_W, out_b):
    node = obs[:, :N * (2 + INPUT_DIM)].reshape(BS, N, 2 + INPUT_DIM)
    pos = node[:, :, :2]
    feats = node[:, :, 2:]
    pos_p = jnp.pad(pos, ((0, 0), (0, NP - N), (0, 0)), constant_values=1e6)
    feats_p = jnp.pad(feats, ((0, 0), (0, NP - N), (0, 2)))
    posT_p = jnp.swapaxes(pos_p, 1, 2)

    agent = jnp.clip(obs[:, -1], 0, N - 1).astype(jnp.int32)
    onehot = jax.nn.one_hot(agent, NP, dtype=jnp.float32)[:, None, :]

    W1 = jnp.pad(enc_W1, ((0, 2), (0, 0)))
    b1 = enc_b1[None, :]
    b2 = enc_b2[None, :]
    bq1 = c1_bq[None, :]; bk1 = c1_bk[None, :]; bv1 = c1_bv[None, :]
    bq2 = c2_bq[None, :]; bk2 = c2_bk[None, :]; bv2 = c2_bv[None, :]
    ob = out_b[None, :]

    def fixed(shape):
        nd = len(shape)
        return pl.BlockSpec(shape, lambda b: (0,) * nd)

    in_specs = [
        pl.BlockSpec((1, NP, 8), lambda b: (b, 0, 0)),
        pl.BlockSpec((1, NP, 2), lambda b: (b, 0, 0)),
        pl.BlockSpec((1, 2, NP), lambda b: (b, 0, 0)),
        pl.BlockSpec((1, 1, NP), lambda b: (b, 0, 0)),
        fixed((8, HIDDEN)), fixed((1, HIDDEN)),
        fixed((HIDDEN, HIDDEN)), fixed((1, HIDDEN)),
        fixed((HIDDEN, D_MODEL)), fixed((1, D_MODEL)),
        fixed((HIDDEN, D_MODEL)), fixed((1, D_MODEL)),
        fixed((HIDDEN, D_MODEL)), fixed((1, D_MODEL)),
        fixed((D_MODEL, D_MODEL)), fixed((1, D_MODEL)),
        fixed((D_MODEL, D_MODEL)), fixed((1, D_MODEL)),
        fixed((D_MODEL, D_MODEL)), fixed((1, D_MODEL)),
        fixed((D_MODEL, OUT_DIM)), fixed((1, OUT_DIM)),
    ]

    out = pl.pallas_call(
        _body,
        grid=(BS,),
        in_specs=in_specs,
        out_specs=pl.BlockSpec((1, 1, OUT_DIM), lambda b: (b, 0, 0)),
        out_shape=jax.ShapeDtypeStruct((BS, 1, OUT_DIM), jnp.float32),
    )(feats_p, pos_p, posT_p, onehot,
      W1, b1, enc_W2, b2,
      c1_Wq, bq1, c1_Wk, bk1, c1_Wv, bv1,
      c2_Wq, bq2, c2_Wk, bk2, c2_Wv, bv2,
      out_W, ob)
    return out[:, 0, :]


# speedup vs baseline: 23.1090x; 1.0251x over previous
import math

import jax
import jax.numpy as jnp
from jax.experimental import pallas as pl

BS = 64
N = 500
NP = 512
INPUT_DIM = 6
HIDDEN = 32
HEADS = 2
OUT_DIM = 10
RADIUS = 0.1
D_MODEL = HIDDEN * HEADS
NEG = -jnp.inf
HE = HIDDEN + 1


def _attn(x_bf, mask_add, wq, bq, wk, bk, wv, bv):
    outs = []
    for h in range(HEADS):
        qh = (jax.lax.dot_general(
            x_bf, wq[h][:], (((1,), (0,)), ((), ())),
            preferred_element_type=jnp.float32) + bq[h][:]).astype(jnp.bfloat16)
        kh = (jax.lax.dot_general(
            x_bf, wk[h][:], (((1,), (0,)), ((), ())),
            preferred_element_type=jnp.float32) + bk[h][:]).astype(jnp.bfloat16)
        ve = (jax.lax.dot_general(
            x_bf, wv[h][:], (((1,), (0,)), ((), ())),
            preferred_element_type=jnp.float32) + bv[h][:]).astype(jnp.bfloat16)
        al = jax.lax.dot_general(
            qh, kh, (((1,), (1,)), ((), ())),
            preferred_element_type=jnp.float32,
        )
        al = al + mask_add
        amax = jnp.max(al, axis=1, keepdims=True)
        amax = jnp.maximum(amax, -1e30)
        ex = jnp.exp(al - amax).astype(jnp.bfloat16)
        oh = jax.lax.dot_general(
            ex, ve, (((1,), (0,)), ((), ())),
            preferred_element_type=jnp.float32,
        )
        den = oh[:, HIDDEN:HE] + 1e-16
        outs.append(oh[:, :HIDDEN] / den)
    return jnp.concatenate(outs, axis=1)


def _body(feats_ref, pos_ref, posT_ref, onehot_ref,
          W1_ref, b1_ref, W2_ref, b2_ref,
          q10_ref, bq10_ref, q11_ref, bq11_ref,
          k10_ref, bk10_ref, k11_ref, bk11_ref,
          v10_ref, bv10_ref, v11_ref, bv11_ref,
          q20_ref, bq20_ref, q21_ref, bq21_ref,
          k20_ref, bk20_ref, k21_ref, bk21_ref,
          v20_ref, bv20_ref, v21_ref, bv21_ref,
          oW_ref, ob_ref, out_ref):
    feats = feats_ref[0]
    h = jnp.maximum(feats @ W1_ref[:] + b1_ref[:], 0.0)
    h = jnp.maximum(h @ W2_ref[:] + b2_ref[:], 0.0)

    pos = pos_ref[0]
    posT = posT_ref[0]
    dx = pos[:, 0:1] - posT[0:1, :]
    dy = pos[:, 1:2] - posT[1:2, :]
    d2 = dx * dx + dy * dy
    ii = jax.lax.broadcasted_iota(jnp.int32, (NP, NP), 0)
    jj = jax.lax.broadcasted_iota(jnp.int32, (NP, NP), 1)
    allowed = (d2 < RADIUS * RADIUS) & (ii != jj)
    mask_add = jnp.where(allowed, 0.0, NEG)

    h = jnp.maximum(
        _attn(h.astype(jnp.bfloat16), mask_add,
              (q10_ref, q11_ref), (bq10_ref, bq11_ref),
              (k10_ref, k11_ref), (bk10_ref, bk11_ref),
              (v10_ref, v11_ref), (bv10_ref, bv11_ref)), 0.0)
    h = jnp.maximum(
        _attn(h.astype(jnp.bfloat16), mask_add,
              (q20_ref, q21_ref), (bq20_ref, bq21_ref),
              (k20_ref, k21_ref), (bk20_ref, bk21_ref),
              (v20_ref, v21_ref), (bv20_ref, bv21_ref)), 0.0)

    emb = jax.lax.dot_general(
        onehot_ref[0], h, (((1,), (0,)), ((), ())),
        preferred_element_type=jnp.float32,
    )
    out_ref[0] = emb @ oW_ref[:] + ob_ref[:]


def _split_heads(Wq, bq, Wk, bk, Wv, bv):
    scale = 1.0 / math.sqrt(HIDDEN)
    out = []
    for h in range(HEADS):
        sl = slice(h * HIDDEN, (h + 1) * HIDDEN)
        wqh = (Wq[:, sl] * scale).astype(jnp.bfloat16)
        bqh = (bq[sl] * scale)[None, :]
        wkh = Wk[:, sl].astype(jnp.bfloat16)
        bkh = bk[sl][None, :]
        wvh = jnp.concatenate(
            [Wv[:, sl], jnp.zeros((Wv.shape[0], 1), jnp.float32)], axis=1
        ).astype(jnp.bfloat16)
        bvh = jnp.concatenate([bv[sl], jnp.ones((1,), jnp.float32)])[None, :]
        out.extend([wqh, bqh, wkh, bkh, wvh, bvh])
    return [out[i] for i in (0, 1, 6, 7, 2, 3, 8, 9, 4, 5, 10, 11)]


@jax.jit
def kernel(obs, enc_W1, enc_b1, enc_W2, enc_b2,
           c1_Wq, c1_bq, c1_Wk, c1_bk, c1_Wv, c1_bv,
           c2_Wq, c2_bq, c2_Wk, c2_bk, c2_Wv, c2_bv,
           out_W, out_b):
    node = obs[:, :N * (2 + INPUT_DIM)].reshape(BS, N, 2 + INPUT_DIM)
    pos = node[:, :, :2]
    feats = node[:, :, 2:]
    pos_p = jnp.pad(pos, ((0, 0), (0, NP - N), (0, 0)), constant_values=1e6)
    feats_p = jnp.pad(feats, ((0, 0), (0, NP - N), (0, 2)))
    posT_p = jnp.swapaxes(pos_p, 1, 2)

    agent = jnp.clip(obs[:, -1], 0, N - 1).astype(jnp.int32)
    onehot = jax.nn.one_hot(agent, NP, dtype=jnp.float32)[:, None, :]

    W1 = jnp.pad(enc_W1, ((0, 2), (0, 0)))
    b1 = enc_b1[None, :]
    b2 = enc_b2[None, :]
    ob = out_b[None, :]
    l1 = _split_heads(c1_Wq, c1_bq, c1_Wk, c1_bk, c1_Wv, c1_bv)
    l2 = _split_heads(c2_Wq, c2_bq, c2_Wk, c2_bk, c2_Wv, c2_bv)

    def fixed(a):
        nd = a.ndim
        return pl.BlockSpec(a.shape, lambda b: (0,) * nd)

    weights = [W1, b1, enc_W2, b2] + l1 + l2 + [out_W, ob]
    in_specs = [
        pl.BlockSpec((1, NP, 8), lambda b: (b, 0, 0)),
        pl.BlockSpec((1, NP, 2), lambda b: (b, 0, 0)),
        pl.BlockSpec((1, 2, NP), lambda b: (b, 0, 0)),
        pl.BlockSpec((1, 1, NP), lambda b: (b, 0, 0)),
    ] + [fixed(w) for w in weights]

    out = pl.pallas_call(
        _body,
        grid=(BS,),
        in_specs=in_specs,
        out_specs=pl.BlockSpec((1, 1, OUT_DIM), lambda b: (b, 0, 0)),
        out_shape=jax.ShapeDtypeStruct((BS, 1, OUT_DIM), jnp.float32),
    )(feats_p, pos_p, posT_p, onehot, *weights)
    return out[:, 0, :]


# speedup vs baseline: 23.2216x; 1.0049x over previous
import math

import jax
import jax.numpy as jnp
from jax.experimental import pallas as pl

BS = 64
N = 500
NP = 512
INPUT_DIM = 6
HIDDEN = 32
HEADS = 2
OUT_DIM = 10
RADIUS = 0.1
D_MODEL = HIDDEN * HEADS
NEG = -jnp.inf
HE = HIDDEN + 1
GPB = 2


def _attn(x_bf, mask_add, wq, bq, wk, bk, wv, bv):
    outs = []
    for h in range(HEADS):
        qh = (jax.lax.dot_general(
            x_bf, wq[h][:], (((1,), (0,)), ((), ())),
            preferred_element_type=jnp.float32) + bq[h][:]).astype(jnp.bfloat16)
        kh = (jax.lax.dot_general(
            x_bf, wk[h][:], (((1,), (0,)), ((), ())),
            preferred_element_type=jnp.float32) + bk[h][:]).astype(jnp.bfloat16)
        ve = (jax.lax.dot_general(
            x_bf, wv[h][:], (((1,), (0,)), ((), ())),
            preferred_element_type=jnp.float32) + bv[h][:]).astype(jnp.bfloat16)
        al = jax.lax.dot_general(
            qh, kh, (((1,), (1,)), ((), ())),
            preferred_element_type=jnp.float32,
        )
        al = al + mask_add
        amax = jnp.max(al, axis=1, keepdims=True)
        amax = jnp.maximum(amax, -1e30)
        ex = jnp.exp(al - amax).astype(jnp.bfloat16)
        oh = jax.lax.dot_general(
            ex, ve, (((1,), (0,)), ((), ())),
            preferred_element_type=jnp.float32,
        )
        den = oh[:, HIDDEN:HE] + 1e-16
        outs.append(oh[:, :HIDDEN] / den)
    return jnp.concatenate(outs, axis=1)


def _body(feats_ref, pos_ref, posT_ref, onehot_ref,
          W1_ref, b1_ref, W2_ref, b2_ref,
          q10_ref, bq10_ref, q11_ref, bq11_ref,
          k10_ref, bk10_ref, k11_ref, bk11_ref,
          v10_ref, bv10_ref, v11_ref, bv11_ref,
          q20_ref, bq20_ref, q21_ref, bq21_ref,
          k20_ref, bk20_ref, k21_ref, bk21_ref,
          v20_ref, bv20_ref, v21_ref, bv21_ref,
          oW_ref, ob_ref, out_ref):
    for g in range(GPB):
        feats = feats_ref[g]
        h = jnp.maximum(feats @ W1_ref[:] + b1_ref[:], 0.0)
        h = jnp.maximum(h @ W2_ref[:] + b2_ref[:], 0.0)

        pos = pos_ref[g]
        posT = posT_ref[g]
        dx = pos[:, 0:1] - posT[0:1, :]
        dy = pos[:, 1:2] - posT[1:2, :]
        d2 = dx * dx + dy * dy
        ii = jax.lax.broadcasted_iota(jnp.int32, (NP, NP), 0)
        jj = jax.lax.broadcasted_iota(jnp.int32, (NP, NP), 1)
        allowed = (d2 < RADIUS * RADIUS) & (ii != jj)
        mask_add = jnp.where(allowed, 0.0, NEG)

        h = jnp.maximum(
            _attn(h.astype(jnp.bfloat16), mask_add,
                  (q10_ref, q11_ref), (bq10_ref, bq11_ref),
                  (k10_ref, k11_ref), (bk10_ref, bk11_ref),
                  (v10_ref, v11_ref), (bv10_ref, bv11_ref)), 0.0)
        h = jnp.maximum(
            _attn(h.astype(jnp.bfloat16), mask_add,
                  (q20_ref, q21_ref), (bq20_ref, bq21_ref),
                  (k20_ref, k21_ref), (bk20_ref, bk21_ref),
                  (v20_ref, v21_ref), (bv20_ref, bv21_ref)), 0.0)

        emb = jax.lax.dot_general(
            onehot_ref[g], h, (((1,), (0,)), ((), ())),
            preferred_element_type=jnp.float32,
        )
        out_ref[g] = emb @ oW_ref[:] + ob_ref[:]


def _split_heads(Wq, bq, Wk, bk, Wv, bv):
    scale = 1.0 / math.sqrt(HIDDEN)
    out = []
    for h in range(HEADS):
        sl = slice(h * HIDDEN, (h + 1) * HIDDEN)
        wqh = (Wq[:, sl] * scale).astype(jnp.bfloat16)
        bqh = (bq[sl] * scale)[None, :]
        wkh = Wk[:, sl].astype(jnp.bfloat16)
        bkh = bk[sl][None, :]
        wvh = jnp.concatenate(
            [Wv[:, sl], jnp.zeros((Wv.shape[0], 1), jnp.float32)], axis=1
        ).astype(jnp.bfloat16)
        bvh = jnp.concatenate([bv[sl], jnp.ones((1,), jnp.float32)])[None, :]
        out.extend([wqh, bqh, wkh, bkh, wvh, bvh])
    return [out[i] for i in (0, 1, 6, 7, 2, 3, 8, 9, 4, 5, 10, 11)]


@jax.jit
def kernel(obs, enc_W1, enc_b1, enc_W2, enc_b2,
           c1_Wq, c1_bq, c1_Wk, c1_bk, c1_Wv, c1_bv,
           c2_Wq, c2_bq, c2_Wk, c2_bk, c2_Wv, c2_bv,
           out_W, out_b):
    node = obs[:, :N * (2 + INPUT_DIM)].reshape(BS, N, 2 + INPUT_DIM)
    pos = node[:, :, :2]
    feats = node[:, :, 2:]
    pos_p = jnp.pad(pos, ((0, 0), (0, NP - N), (0, 0)), constant_values=1e6)
    feats_p = jnp.pad(feats, ((0, 0), (0, NP - N), (0, 2)))
    posT_p = jnp.swapaxes(pos_p, 1, 2)

    agent = jnp.clip(obs[:, -1], 0, N - 1).astype(jnp.int32)
    onehot = jax.nn.one_hot(agent, NP, dtype=jnp.float32)[:, None, :]

    W1 = jnp.pad(enc_W1, ((0, 2), (0, 0)))
    b1 = enc_b1[None, :]
    b2 = enc_b2[None, :]
    ob = out_b[None, :]
    l1 = _split_heads(c1_Wq, c1_bq, c1_Wk, c1_bk, c1_Wv, c1_bv)
    l2 = _split_heads(c2_Wq, c2_bq, c2_Wk, c2_bk, c2_Wv, c2_bv)

    def fixed(a):
        nd = a.ndim
        return pl.BlockSpec(a.shape, lambda b: (0,) * nd)

    weights = [W1, b1, enc_W2, b2] + l1 + l2 + [out_W, ob]
    in_specs = [
        pl.BlockSpec((GPB, NP, 8), lambda b: (b, 0, 0)),
        pl.BlockSpec((GPB, NP, 2), lambda b: (b, 0, 0)),
        pl.BlockSpec((GPB, 2, NP), lambda b: (b, 0, 0)),
        pl.BlockSpec((GPB, 1, NP), lambda b: (b, 0, 0)),
    ] + [fixed(w) for w in weights]

    out = pl.pallas_call(
        _body,
        grid=(BS // GPB,),
        in_specs=in_specs,
        out_specs=pl.BlockSpec((GPB, 1, OUT_DIM), lambda b: (b, 0, 0)),
        out_shape=jax.ShapeDtypeStruct((BS, 1, OUT_DIM), jnp.float32),
    )(feats_p, pos_p, posT_p, onehot, *weights)
    return out[:, 0, :]


# speedup vs baseline: 23.4411x; 1.0094x over previous
import math

import jax
import jax.numpy as jnp
from jax.experimental import pallas as pl

BS = 64
N = 500
NP = 512
INPUT_DIM = 6
HIDDEN = 32
HEADS = 2
OUT_DIM = 10
RADIUS = 0.1
D_MODEL = HIDDEN * HEADS
NEG = -jnp.inf
HE = HIDDEN + 1
GPB = 2


def _attn(x_bf, mask_add, wq, bq, wk, bk, wv, bv):
    outs = []
    for h in range(HEADS):
        qh = (jax.lax.dot_general(
            x_bf, wq[h][:], (((1,), (0,)), ((), ())),
            preferred_element_type=jnp.float32) + bq[h][:]).astype(jnp.bfloat16)
        kh = (jax.lax.dot_general(
            x_bf, wk[h][:], (((1,), (0,)), ((), ())),
            preferred_element_type=jnp.float32) + bk[h][:]).astype(jnp.bfloat16)
        ve = (jax.lax.dot_general(
            x_bf, wv[h][:], (((1,), (0,)), ((), ())),
            preferred_element_type=jnp.float32) + bv[h][:]).astype(jnp.bfloat16)
        al = jax.lax.dot_general(
            qh, kh, (((1,), (1,)), ((), ())),
            preferred_element_type=jnp.float32,
        ).astype(jnp.bfloat16)
        al = al + mask_add
        amax = jnp.max(al, axis=1, keepdims=True)
        amax = jnp.maximum(amax, jnp.bfloat16(-1e30))
        ex = jnp.exp(al - amax)
        oh = jax.lax.dot_general(
            ex, ve, (((1,), (0,)), ((), ())),
            preferred_element_type=jnp.float32,
        )
        den = oh[:, HIDDEN:HE] + 1e-16
        outs.append(oh[:, :HIDDEN] / den)
    return jnp.concatenate(outs, axis=1)


def _body(feats_ref, pos_ref, posT_ref, onehot_ref,
          W1_ref, b1_ref, W2_ref, b2_ref,
          q10_ref, bq10_ref, q11_ref, bq11_ref,
          k10_ref, bk10_ref, k11_ref, bk11_ref,
          v10_ref, bv10_ref, v11_ref, bv11_ref,
          q20_ref, bq20_ref, q21_ref, bq21_ref,
          k20_ref, bk20_ref, k21_ref, bk21_ref,
          v20_ref, bv20_ref, v21_ref, bv21_ref,
          oW_ref, ob_ref, out_ref):
    for g in range(GPB):
        feats = feats_ref[g]
        h = jnp.maximum(feats @ W1_ref[:] + b1_ref[:], 0.0)
        h = jnp.maximum(h @ W2_ref[:] + b2_ref[:], 0.0)

        pos = pos_ref[g]
        posT = posT_ref[g]
        dx = pos[:, 0:1] - posT[0:1, :]
        dy = pos[:, 1:2] - posT[1:2, :]
        d2 = dx * dx + dy * dy
        ii = jax.lax.broadcasted_iota(jnp.int32, (NP, NP), 0)
        jj = jax.lax.broadcasted_iota(jnp.int32, (NP, NP), 1)
        allowed = (d2 < RADIUS * RADIUS) & (ii != jj)
        mask_add = jnp.where(allowed, 0.0, NEG).astype(jnp.bfloat16)

        h = jnp.maximum(
            _attn(h.astype(jnp.bfloat16), mask_add,
                  (q10_ref, q11_ref), (bq10_ref, bq11_ref),
                  (k10_ref, k11_ref), (bk10_ref, bk11_ref),
                  (v10_ref, v11_ref), (bv10_ref, bv11_ref)), 0.0)
        h = jnp.maximum(
            _attn(h.astype(jnp.bfloat16), mask_add,
                  (q20_ref, q21_ref), (bq20_ref, bq21_ref),
                  (k20_ref, k21_ref), (bk20_ref, bk21_ref),
                  (v20_ref, v21_ref), (bv20_ref, bv21_ref)), 0.0)

        emb = jax.lax.dot_general(
            onehot_ref[g], h, (((1,), (0,)), ((), ())),
            preferred_element_type=jnp.float32,
        )
        out_ref[g] = emb @ oW_ref[:] + ob_ref[:]


def _split_heads(Wq, bq, Wk, bk, Wv, bv):
    scale = 1.0 / math.sqrt(HIDDEN)
    out = []
    for h in range(HEADS):
        sl = slice(h * HIDDEN, (h + 1) * HIDDEN)
        wqh = (Wq[:, sl] * scale).astype(jnp.bfloat16)
        bqh = (bq[sl] * scale)[None, :]
        wkh = Wk[:, sl].astype(jnp.bfloat16)
        bkh = bk[sl][None, :]
        wvh = jnp.concatenate(
            [Wv[:, sl], jnp.zeros((Wv.shape[0], 1), jnp.float32)], axis=1
        ).astype(jnp.bfloat16)
        bvh = jnp.concatenate([bv[sl], jnp.ones((1,), jnp.float32)])[None, :]
        out.extend([wqh, bqh, wkh, bkh, wvh, bvh])
    return [out[i] for i in (0, 1, 6, 7, 2, 3, 8, 9, 4, 5, 10, 11)]


@jax.jit
def kernel(obs, enc_W1, enc_b1, enc_W2, enc_b2,
           c1_Wq, c1_bq, c1_Wk, c1_bk, c1_Wv, c1_bv,
           c2_Wq, c2_bq, c2_Wk, c2_bk, c2_Wv, c2_bv,
           out_W, out_b):
    node = obs[:, :N * (2 + INPUT_DIM)].reshape(BS, N, 2 + INPUT_DIM)
    pos = node[:, :, :2]
    feats = node[:, :, 2:]
    pos_p = jnp.pad(pos, ((0, 0), (0, NP - N), (0, 0)), constant_values=1e6)
    feats_p = jnp.pad(feats, ((0, 0), (0, NP - N), (0, 2)))
    posT_p = jnp.swapaxes(pos_p, 1, 2)

    agent = jnp.clip(obs[:, -1], 0, N - 1).astype(jnp.int32)
    onehot = jax.nn.one_hot(agent, NP, dtype=jnp.float32)[:, None, :]

    W1 = jnp.pad(enc_W1, ((0, 2), (0, 0)))
    b1 = enc_b1[None, :]
    b2 = enc_b2[None, :]
    ob = out_b[None, :]
    l1 = _split_heads(c1_Wq, c1_bq, c1_Wk, c1_bk, c1_Wv, c1_bv)
    l2 = _split_heads(c2_Wq, c2_bq, c2_Wk, c2_bk, c2_Wv, c2_bv)

    def fixed(a):
        nd = a.ndim
        return pl.BlockSpec(a.shape, lambda b: (0,) * nd)

    weights = [W1, b1, enc_W2, b2] + l1 + l2 + [out_W, ob]
    in_specs = [
        pl.BlockSpec((GPB, NP, 8), lambda b: (b, 0, 0)),
        pl.BlockSpec((GPB, NP, 2), lambda b: (b, 0, 0)),
        pl.BlockSpec((GPB, 2, NP), lambda b: (b, 0, 0)),
        pl.BlockSpec((GPB, 1, NP), lambda b: (b, 0, 0)),
    ] + [fixed(w) for w in weights]

    out = pl.pallas_call(
        _body,
        grid=(BS // GPB,),
        in_specs=in_specs,
        out_specs=pl.BlockSpec((GPB, 1, OUT_DIM), lambda b: (b, 0, 0)),
        out_shape=jax.ShapeDtypeStruct((BS, 1, OUT_DIM), jnp.float32),
    )(feats_p, pos_p, posT_p, onehot, *weights)
    return out[:, 0, :]
